# initial kernel scaffold (unmeasured)
import jax
import jax.numpy as jnp
from jax import lax
from jax.experimental import pallas as pl
from jax.experimental.pallas import tpu as pltpu

N_DEV = 4


def kernel(x, w_mat):
    k_full, k_per = x.shape
    _, n = w_mat.shape
    m_per = k_full // N_DEV
    assert m_per == k_per, (x.shape, w_mat.shape)

    x = x.astype(jnp.bfloat16)
    w_mat = w_mat.astype(jnp.bfloat16)

    def body(x_ref, w_ref, out_ref, comm_ref, send_sems, recv_sems):
        my = lax.axis_index("i")

        barrier = pltpu.get_barrier_semaphore()
        for d in range(1, N_DEV):
            pl.semaphore_signal(
                barrier,
                inc=1,
                device_id=((my + d) % N_DEV,),
                device_id_type=pl.DeviceIdType.MESH,
            )
        pl.semaphore_wait(barrier, N_DEV - 1)

        sends = []
        for d in range(1, N_DEV):
            tgt = (my + d) % N_DEV
            rdma = pltpu.make_async_remote_copy(
                src_ref=x_ref.at[pl.ds(tgt * m_per, m_per), :],
                dst_ref=comm_ref.at[my],
                send_sem=send_sems.at[d - 1],
                recv_sem=recv_sems.at[my],
                device_id=(tgt,),
                device_id_type=pl.DeviceIdType.MESH,
            )
            rdma.start()
            sends.append(rdma)

        acc = jnp.dot(
            x_ref[pl.ds(my * m_per, m_per), :],
            w_ref[pl.ds(my * m_per, m_per), :],
            preferred_element_type=jnp.float32,
        )

        for d in (1, 3, 2):
            src = (my + d) % N_DEV
            recv = pltpu.make_async_remote_copy(
                src_ref=comm_ref.at[src],
                dst_ref=comm_ref.at[src],
                send_sem=send_sems.at[0],
                recv_sem=recv_sems.at[src],
                device_id=(my,),
                device_id_type=pl.DeviceIdType.MESH,
            )
            recv.wait_recv()
            acc = acc + jnp.dot(
                comm_ref[src],
                w_ref[pl.ds(src * m_per, m_per), :],
                preferred_element_type=jnp.float32,
            )

        out_ref[:, :] = acc

        for rdma in sends:
            rdma.wait_send()

    return pl.pallas_call(
        body,
        out_shape=jax.ShapeDtypeStruct((m_per, n), jnp.float32),
        in_specs=[
            pl.BlockSpec(memory_space=pltpu.VMEM),
            pl.BlockSpec(memory_space=pltpu.VMEM),
        ],
        out_specs=pl.BlockSpec(memory_space=pltpu.VMEM),
        scratch_shapes=[
            pltpu.VMEM((N_DEV, m_per, k_per), jnp.bfloat16),
            pltpu.SemaphoreType.DMA((N_DEV - 1,)),
            pltpu.SemaphoreType.DMA((N_DEV,)),
        ],
        compiler_params=pltpu.CompilerParams(collective_id=0),
    )(x, w_mat)


# baseline (device time: 91831 ns/iter reference)
import jax
import jax.numpy as jnp
from jax import lax
from jax.experimental import pallas as pl
from jax.experimental.pallas import tpu as pltpu

N_DEV = 4


def kernel(x, w_mat):
    k_full, k_per = x.shape
    _, n = w_mat.shape
    m_per = k_full // N_DEV
    n_half = n // 2
    assert m_per == k_per, (x.shape, w_mat.shape)

    x = x.astype(jnp.bfloat16)
    w_mat = w_mat.astype(jnp.bfloat16)

    def body(x_hbm, w_hbm, out_ref, comm_ref, wbuf, send_sems, recv_sems,
             wsems, xsem):
        my = lax.axis_index("i")

        barrier = pltpu.get_barrier_semaphore()
        for d in range(1, N_DEV):
            pl.semaphore_signal(
                barrier,
                inc=1,
                device_id=((my + d) % N_DEV,),
                device_id_type=pl.DeviceIdType.MESH,
            )
        pl.semaphore_wait(barrier, N_DEV - 1)

        sends = []
        for d in range(1, N_DEV):
            tgt = (my + d) % N_DEV
            rdma = pltpu.make_async_remote_copy(
                src_ref=x_hbm.at[pl.ds(tgt * m_per, m_per), :],
                dst_ref=comm_ref.at[my],
                send_sem=send_sems.at[d - 1],
                recv_sem=recv_sems.at[my],
                device_id=(tgt,),
                device_id_type=pl.DeviceIdType.MESH,
            )
            rdma.start()
            sends.append(rdma)

        xcpy = pltpu.make_async_copy(
            x_hbm.at[pl.ds(my * m_per, m_per), :], comm_ref.at[my], xsem
        )
        xcpy.start()

        order = [my, (my + 1) % N_DEV, (my + 3) % N_DEV, (my + 2) % N_DEV]

        wcopies = []
        for i, kblk in enumerate(order):
            wcopies.append(
                pltpu.make_async_copy(
                    w_hbm.at[pl.ds(kblk * m_per, m_per), :],
                    wbuf.at[i % 2],
                    wsems.at[i % 2],
                )
            )
        wcopies[0].start()
        wcopies[1].start()

        for i, src in enumerate(order):
            if i == 0:
                xcpy.wait()
            else:
                recv = pltpu.make_async_remote_copy(
                    src_ref=comm_ref.at[src],
                    dst_ref=comm_ref.at[src],
                    send_sem=send_sems.at[0],
                    recv_sem=recv_sems.at[src],
                    device_id=(my,),
                    device_id_type=pl.DeviceIdType.MESH,
                )
                recv.wait_recv()
            wcopies[i].wait()
            if i + 1 < N_DEV and i >= 1:
                wcopies[i + 1].start()
            for h in range(2):
                sl = pl.ds(h * n_half, n_half)
                partial = jnp.dot(
                    comm_ref[src],
                    wbuf[i % 2, :, sl],
                    preferred_element_type=jnp.float32,
                )
                if i == 0:
                    out_ref[:, sl] = partial
                else:
                    out_ref[:, sl] = out_ref[:, sl] + partial

        for rdma in sends:
            rdma.wait_send()

    return pl.pallas_call(
        body,
        out_shape=jax.ShapeDtypeStruct((m_per, n), jnp.float32),
        in_specs=[
            pl.BlockSpec(memory_space=pl.ANY),
            pl.BlockSpec(memory_space=pl.ANY),
        ],
        out_specs=pl.BlockSpec(memory_space=pltpu.VMEM),
        scratch_shapes=[
            pltpu.VMEM((N_DEV, m_per, k_per), jnp.bfloat16),
            pltpu.VMEM((2, m_per, n), jnp.bfloat16),
            pltpu.SemaphoreType.DMA((N_DEV - 1,)),
            pltpu.SemaphoreType.DMA((N_DEV,)),
            pltpu.SemaphoreType.DMA((2,)),
            pltpu.SemaphoreType.DMA,
        ],
        compiler_params=pltpu.CompilerParams(collective_id=0),
    )(x, w_mat)


# device time: 81116 ns/iter; 1.1321x vs baseline; 1.1321x over previous
import jax
import jax.numpy as jnp
from jax import lax
from jax.experimental import pallas as pl
from jax.experimental.pallas import tpu as pltpu

N_DEV = 4


def kernel(x, w_mat):
    k_full, k_per = x.shape
    _, n = w_mat.shape
    m_per = k_full // N_DEV
    n_half = n // 2
    assert m_per == k_per, (x.shape, w_mat.shape)

    x = x.astype(jnp.bfloat16)

    def body(x_hbm, w_hbm, out_ref, comm_ref, wbuf, send_sems, recv_sems,
             wsem, xsem):
        my = lax.axis_index("i")

        barrier = pltpu.get_barrier_semaphore()
        for d in range(1, N_DEV):
            pl.semaphore_signal(
                barrier,
                inc=1,
                device_id=((my + d) % N_DEV,),
                device_id_type=pl.DeviceIdType.MESH,
            )
        pl.semaphore_wait(barrier, N_DEV - 1)

        sends = []
        for d in range(1, N_DEV):
            tgt = (my + d) % N_DEV
            rdma = pltpu.make_async_remote_copy(
                src_ref=x_hbm.at[pl.ds(tgt * m_per, m_per), :],
                dst_ref=comm_ref.at[my],
                send_sem=send_sems.at[d - 1],
                recv_sem=recv_sems.at[my],
                device_id=(tgt,),
                device_id_type=pl.DeviceIdType.MESH,
            )
            rdma.start()
            sends.append(rdma)

        xcpy = pltpu.make_async_copy(
            x_hbm.at[pl.ds(my * m_per, m_per), :], comm_ref.at[my], xsem
        )
        xcpy.start()

        order = [my, (my + 1) % N_DEV, (my + 3) % N_DEV, (my + 2) % N_DEV]

        wcopies = [
            pltpu.make_async_copy(
                w_hbm.at[pl.ds(kblk * m_per, m_per), :], wbuf, wsem
            )
            for kblk in order
        ]
        wcopies[0].start()

        for i, src in enumerate(order):
            if i == 0:
                xcpy.wait()
            else:
                recv = pltpu.make_async_remote_copy(
                    src_ref=comm_ref.at[src],
                    dst_ref=comm_ref.at[src],
                    send_sem=send_sems.at[0],
                    recv_sem=recv_sems.at[src],
                    device_id=(my,),
                    device_id_type=pl.DeviceIdType.MESH,
                )
                recv.wait_recv()
            wcopies[i].wait()
            for h in range(2):
                sl = pl.ds(h * n_half, n_half)
                partial = jnp.dot(
                    comm_ref[src],
                    wbuf[:, sl].astype(jnp.bfloat16),
                    preferred_element_type=jnp.float32,
                )
                if i == 0:
                    out_ref[:, sl] = partial
                else:
                    out_ref[:, sl] = out_ref[:, sl] + partial
            if i + 1 < N_DEV:
                wcopies[i + 1].start()

        for rdma in sends:
            rdma.wait_send()

    return pl.pallas_call(
        body,
        out_shape=jax.ShapeDtypeStruct((m_per, n), jnp.float32),
        in_specs=[
            pl.BlockSpec(memory_space=pl.ANY),
            pl.BlockSpec(memory_space=pl.ANY),
        ],
        out_specs=pl.BlockSpec(memory_space=pltpu.VMEM),
        scratch_shapes=[
            pltpu.VMEM((N_DEV, m_per, k_per), jnp.bfloat16),
            pltpu.VMEM((m_per, n), jnp.float32),
            pltpu.SemaphoreType.DMA((N_DEV - 1,)),
            pltpu.SemaphoreType.DMA((N_DEV,)),
            pltpu.SemaphoreType.DMA,
            pltpu.SemaphoreType.DMA,
        ],
        compiler_params=pltpu.CompilerParams(collective_id=0),
    )(x, w_mat)


# device time: 74939 ns/iter; 1.2254x vs baseline; 1.0824x over previous
import jax
import jax.numpy as jnp
from jax import lax
from jax.experimental import pallas as pl
from jax.experimental.pallas import tpu as pltpu

N_DEV = 4


def kernel(x, w_mat):
    k_full, k_per = x.shape
    _, n = w_mat.shape
    m_per = k_full // N_DEV
    n_half = n // 2
    assert m_per == k_per, (x.shape, w_mat.shape)

    x = x.astype(jnp.bfloat16)

    def body(x_hbm, w_hbm, out_ref, comm_ref, wbuf, send_sems, recv_sems,
             wsem, xsem):
        my = lax.axis_index("i")

        barrier = pltpu.get_barrier_semaphore()
        for d in range(1, N_DEV):
            pl.semaphore_signal(
                barrier,
                inc=1,
                device_id=((my + d) % N_DEV,),
                device_id_type=pl.DeviceIdType.MESH,
            )
        pl.semaphore_wait(barrier, N_DEV - 1)

        sends = []
        for d in range(1, N_DEV):
            tgt = (my + d) % N_DEV
            rdma = pltpu.make_async_remote_copy(
                src_ref=x_hbm.at[pl.ds(tgt * m_per, m_per), :],
                dst_ref=comm_ref.at[my],
                send_sem=send_sems.at[d - 1],
                recv_sem=recv_sems.at[my],
                device_id=(tgt,),
                device_id_type=pl.DeviceIdType.MESH,
            )
            rdma.start()
            sends.append(rdma)

        xcpy = pltpu.make_async_copy(
            x_hbm.at[pl.ds(my * m_per, m_per), :], comm_ref.at[my], xsem
        )
        xcpy.start()

        order = [my, (my + 1) % N_DEV, (my + 3) % N_DEV, (my + 2) % N_DEV]

        wcopies = [
            pltpu.make_async_copy(
                w_hbm.at[pl.ds(kblk * m_per, m_per),
                         pl.ds(h * n_half, n_half)],
                wbuf.at[ci % 2],
                wsem.at[ci % 2],
            )
            for ci, (kblk, h) in enumerate(
                (kblk, h) for kblk in order for h in range(2)
            )
        ]
        wcopies[0].start()
        wcopies[1].start()

        for i, src in enumerate(order):
            if i == 0:
                xcpy.wait()
            else:
                recv = pltpu.make_async_remote_copy(
                    src_ref=comm_ref.at[src],
                    dst_ref=comm_ref.at[src],
                    send_sem=send_sems.at[0],
                    recv_sem=recv_sems.at[src],
                    device_id=(my,),
                    device_id_type=pl.DeviceIdType.MESH,
                )
                recv.wait_recv()
            for h in range(2):
                ci = 2 * i + h
                sl = pl.ds(h * n_half, n_half)
                wcopies[ci].wait()
                partial = jnp.dot(
                    comm_ref[src],
                    wbuf[ci % 2].astype(jnp.bfloat16),
                    preferred_element_type=jnp.float32,
                )
                if i == 0:
                    out_ref[:, sl] = partial
                else:
                    out_ref[:, sl] = out_ref[:, sl] + partial
                if ci + 2 < 2 * N_DEV:
                    wcopies[ci + 2].start()

        for rdma in sends:
            rdma.wait_send()

    return pl.pallas_call(
        body,
        out_shape=jax.ShapeDtypeStruct((m_per, n), jnp.float32),
        in_specs=[
            pl.BlockSpec(memory_space=pl.ANY),
            pl.BlockSpec(memory_space=pl.ANY),
        ],
        out_specs=pl.BlockSpec(memory_space=pltpu.VMEM),
        scratch_shapes=[
            pltpu.VMEM((N_DEV, m_per, k_per), jnp.bfloat16),
            pltpu.VMEM((2, m_per, n_half), jnp.float32),
            pltpu.SemaphoreType.DMA((N_DEV - 1,)),
            pltpu.SemaphoreType.DMA((N_DEV,)),
            pltpu.SemaphoreType.DMA((2,)),
            pltpu.SemaphoreType.DMA,
        ],
        compiler_params=pltpu.CompilerParams(collective_id=0),
    )(x, w_mat)


# device time: 73890 ns/iter; 1.2428x vs baseline; 1.0142x over previous
import jax
import jax.numpy as jnp
from jax import lax
from jax.experimental import pallas as pl
from jax.experimental.pallas import tpu as pltpu

N_DEV = 4
N_MH = 2


def kernel(x, w_mat):
    k_full, k_per = x.shape
    _, n = w_mat.shape
    m_per = k_full // N_DEV
    m_half = m_per // N_MH
    n_half = n // 2
    assert m_per == k_per, (x.shape, w_mat.shape)

    x = x.astype(jnp.bfloat16)

    def body(x_hbm, w_hbm, out_ref, comm_ref, wbuf, wbf, send_sems,
             recv_sems, wsem, xsem):
        my = lax.axis_index("i")

        barrier = pltpu.get_barrier_semaphore()
        for d in range(1, N_DEV):
            pl.semaphore_signal(
                barrier,
                inc=1,
                device_id=((my + d) % N_DEV,),
                device_id_type=pl.DeviceIdType.MESH,
            )
        pl.semaphore_wait(barrier, N_DEV - 1)

        sends = []
        for d in range(1, N_DEV):
            tgt = (my + d) % N_DEV
            for mh in range(N_MH):
                rdma = pltpu.make_async_remote_copy(
                    src_ref=x_hbm.at[
                        pl.ds(tgt * m_per + mh * m_half, m_half), :],
                    dst_ref=comm_ref.at[my, pl.ds(mh * m_half, m_half), :],
                    send_sem=send_sems.at[len(sends)],
                    recv_sem=recv_sems.at[my, mh],
                    device_id=(tgt,),
                    device_id_type=pl.DeviceIdType.MESH,
                )
                rdma.start()
                sends.append(rdma)

        xcpy = pltpu.make_async_copy(
            x_hbm.at[pl.ds(my * m_per, m_per), :], comm_ref.at[my], xsem
        )
        xcpy.start()

        order = [my, (my + 1) % N_DEV, (my + 3) % N_DEV, (my + 2) % N_DEV]

        wcopies = [
            pltpu.make_async_copy(
                w_hbm.at[pl.ds(kblk * m_per, m_per),
                         pl.ds(h * n_half, n_half)],
                wbuf.at[ci % 2],
                wsem.at[ci % 2],
            )
            for ci, (kblk, h) in enumerate(
                (kblk, h) for kblk in order for h in range(2)
            )
        ]
        wcopies[0].start()
        wcopies[1].start()

        for i, src in enumerate(order):
            for h in range(2):
                ci = 2 * i + h
                wcopies[ci].wait()
                wbf[h] = wbuf[ci % 2].astype(jnp.bfloat16)
                if ci + 2 < 2 * N_DEV:
                    wcopies[ci + 2].start()

            for mh in range(N_MH):
                if i == 0:
                    if mh == 0:
                        xcpy.wait()
                else:
                    recv = pltpu.make_async_remote_copy(
                        src_ref=comm_ref.at[src, pl.ds(mh * m_half,
                                                       m_half), :],
                        dst_ref=comm_ref.at[src, pl.ds(mh * m_half,
                                                       m_half), :],
                        send_sem=send_sems.at[0],
                        recv_sem=recv_sems.at[src, mh],
                        device_id=(my,),
                        device_id_type=pl.DeviceIdType.MESH,
                    )
                    recv.wait_recv()
                for h in range(2):
                    msl = pl.ds(mh * m_half, m_half)
                    nsl = pl.ds(h * n_half, n_half)
                    partial = jnp.dot(
                        comm_ref[src, msl, :],
                        wbf[h],
                        preferred_element_type=jnp.float32,
                    )
                    if i == 0:
                        out_ref[msl, nsl] = partial
                    else:
                        out_ref[msl, nsl] = out_ref[msl, nsl] + partial

        for rdma in sends:
            rdma.wait_send()

    return pl.pallas_call(
        body,
        out_shape=jax.ShapeDtypeStruct((m_per, n), jnp.float32),
        in_specs=[
            pl.BlockSpec(memory_space=pl.ANY),
            pl.BlockSpec(memory_space=pl.ANY),
        ],
        out_specs=pl.BlockSpec(memory_space=pltpu.VMEM),
        scratch_shapes=[
            pltpu.VMEM((N_DEV, m_per, k_per), jnp.bfloat16),
            pltpu.VMEM((2, m_per, n_half), jnp.float32),
            pltpu.VMEM((2, m_per, n_half), jnp.bfloat16),
            pltpu.SemaphoreType.DMA(((N_DEV - 1) * N_MH,)),
            pltpu.SemaphoreType.DMA((N_DEV, N_MH)),
            pltpu.SemaphoreType.DMA((2,)),
            pltpu.SemaphoreType.DMA,
        ],
        compiler_params=pltpu.CompilerParams(collective_id=0),
    )(x, w_mat)


# device time: 73362 ns/iter; 1.2518x vs baseline; 1.0072x over previous
import jax
import jax.numpy as jnp
from jax import lax
from jax.experimental import pallas as pl
from jax.experimental.pallas import tpu as pltpu

N_DEV = 4
N_Q = 4


def kernel(x, w_mat):
    k_full, k_per = x.shape
    _, n = w_mat.shape
    m_per = k_full // N_DEV
    m_q = m_per // N_Q
    n_half = n // 2
    assert m_per == k_per, (x.shape, w_mat.shape)

    def body(x_hbm, w_hbm, out_hbm, comm_ref, sendbuf, xstage, wbuf, wbf,
             acc, send_sems, recv_sems, wsem, xsem, osem):
        my = lax.axis_index("i")
        right = (my + 1) % N_DEV
        left = (my + 3) % N_DEV
        diag = (my + 2) % N_DEV

        barrier = pltpu.get_barrier_semaphore()
        for d in range(1, N_DEV):
            pl.semaphore_signal(
                barrier,
                inc=1,
                device_id=((my + d) % N_DEV,),
                device_id_type=pl.DeviceIdType.MESH,
            )

        conv_order = [right, left, diag, my]
        xcopies = [
            pltpu.make_async_copy(
                x_hbm.at[pl.ds(blk * m_per, m_per), :],
                xstage.at[b % 2],
                xsem.at[b % 2],
            )
            for b, blk in enumerate(conv_order)
        ]
        xcopies[0].start()
        xcopies[1].start()

        proc_order = [my, right, left, diag]
        wcopies = [
            pltpu.make_async_copy(
                w_hbm.at[pl.ds(kblk * m_per, m_per),
                         pl.ds(h * n_half, n_half)],
                wbuf.at[ci % 2],
                wsem.at[ci % 2],
            )
            for ci, (kblk, h) in enumerate(
                (kblk, h) for kblk in proc_order for h in range(2)
            )
        ]
        wcopies[0].start()
        wcopies[1].start()

        pl.semaphore_wait(barrier, N_DEV - 1)

        sends = []
        for b in range(N_DEV - 1):
            tgt = conv_order[b]
            xcopies[b].wait()
            sendbuf[b] = xstage[b % 2].astype(jnp.bfloat16)
            if b + 2 < N_DEV:
                xcopies[b + 2].start()
            for q in range(N_Q):
                qsl = pl.ds(q * m_q, m_q)
                rdma = pltpu.make_async_remote_copy(
                    src_ref=sendbuf.at[b, qsl, :],
                    dst_ref=comm_ref.at[my, qsl, :],
                    send_sem=send_sems.at[len(sends)],
                    recv_sem=recv_sems.at[my, q],
                    device_id=(tgt,),
                    device_id_type=pl.DeviceIdType.MESH,
                )
                rdma.start()
                sends.append(rdma)
        xcopies[3].wait()
        comm_ref[my] = xstage[1].astype(jnp.bfloat16)

        outcopies = [
            pltpu.make_async_copy(
                acc.at[pl.ds(q * m_q, m_q), :],
                out_hbm.at[pl.ds(q * m_q, m_q), :],
                osem.at[q],
            )
            for q in range(N_Q)
        ]
        for i, src in enumerate(proc_order):
            for h in range(2):
                ci = 2 * i + h
                wcopies[ci].wait()
                wbf[h] = wbuf[ci % 2].astype(jnp.bfloat16)
                if ci + 2 < 2 * N_DEV:
                    wcopies[ci + 2].start()
            for q in range(N_Q):
                qsl = pl.ds(q * m_q, m_q)
                if i > 0:
                    recv = pltpu.make_async_remote_copy(
                        src_ref=comm_ref.at[src, qsl, :],
                        dst_ref=comm_ref.at[src, qsl, :],
                        send_sem=send_sems.at[0],
                        recv_sem=recv_sems.at[src, q],
                        device_id=(my,),
                        device_id_type=pl.DeviceIdType.MESH,
                    )
                    recv.wait_recv()
                for h in range(2):
                    nsl = pl.ds(h * n_half, n_half)
                    partial = jnp.dot(
                        comm_ref[src, qsl, :],
                        wbf[h],
                        preferred_element_type=jnp.float32,
                    )
                    if i == 0:
                        acc[qsl, nsl] = partial
                    else:
                        acc[qsl, nsl] = acc[qsl, nsl] + partial
                if i == N_DEV - 1:
                    outcopies[q].start()

        for oc in outcopies:
            oc.wait()
        for rdma in sends:
            rdma.wait_send()

    return pl.pallas_call(
        body,
        out_shape=jax.ShapeDtypeStruct((m_per, n), jnp.float32),
        in_specs=[
            pl.BlockSpec(memory_space=pl.ANY),
            pl.BlockSpec(memory_space=pl.ANY),
        ],
        out_specs=pl.BlockSpec(memory_space=pl.ANY),
        scratch_shapes=[
            pltpu.VMEM((N_DEV, m_per, k_per), jnp.bfloat16),
            pltpu.VMEM((N_DEV - 1, m_per, k_per), jnp.bfloat16),
            pltpu.VMEM((2, m_per, k_per), jnp.float32),
            pltpu.VMEM((2, m_per, n_half), jnp.float32),
            pltpu.VMEM((2, m_per, n_half), jnp.bfloat16),
            pltpu.VMEM((m_per, n), jnp.float32),
            pltpu.SemaphoreType.DMA(((N_DEV - 1) * N_Q,)),
            pltpu.SemaphoreType.DMA((N_DEV, N_Q)),
            pltpu.SemaphoreType.DMA((2,)),
            pltpu.SemaphoreType.DMA((2,)),
            pltpu.SemaphoreType.DMA((N_Q,)),
        ],
        compiler_params=pltpu.CompilerParams(
            collective_id=0,
            vmem_limit_bytes=100 * 1024 * 1024,
        ),
    )(x, w_mat)


# device time: 63796 ns/iter; 1.4394x vs baseline; 1.1499x over previous
import jax
import jax.numpy as jnp
from jax import lax
from jax.experimental import pallas as pl
from jax.experimental.pallas import tpu as pltpu

N_DEV = 4
N_Q = 4


def kernel(x, w_mat):
    k_full, k_per = x.shape
    _, n = w_mat.shape
    m_per = k_full // N_DEV
    m_q = m_per // N_Q
    n_half = n // 2
    assert m_per == k_per, (x.shape, w_mat.shape)

    def body(x_hbm, w_hbm, out_hbm, comm_ref, sendbuf, xstage, wbuf, wbf,
             acc, send_sems, recv_sems, wsem, xsem, osem):
        my = lax.axis_index("i")
        right = (my + 1) % N_DEV
        left = (my + 3) % N_DEV
        diag = (my + 2) % N_DEV

        barrier = pltpu.get_barrier_semaphore()
        for d in range(1, N_DEV):
            pl.semaphore_signal(
                barrier,
                inc=1,
                device_id=((my + d) % N_DEV,),
                device_id_type=pl.DeviceIdType.MESH,
            )

        conv_order = [diag, right, left, my]
        xcopies = [
            pltpu.make_async_copy(
                x_hbm.at[pl.ds(blk * m_per, m_per), :],
                xstage.at[b % 2],
                xsem.at[b % 2],
            )
            for b, blk in enumerate(conv_order)
        ]
        xcopies[0].start()
        xcopies[1].start()

        proc_order = [my, right, left, diag]
        wcopies = [
            pltpu.make_async_copy(
                w_hbm.at[pl.ds(kblk * m_per, m_per),
                         pl.ds(h * n_half, n_half)],
                wbuf.at[ci % 2],
                wsem.at[ci % 2],
            )
            for ci, (kblk, h) in enumerate(
                (kblk, h) for kblk in proc_order for h in range(2)
            )
        ]
        wcopies[0].start()
        wcopies[1].start()

        pl.semaphore_wait(barrier, N_DEV - 1)

        sends = []
        for b in range(N_DEV - 1):
            tgt = conv_order[b]
            xcopies[b].wait()
            sendbuf[b] = xstage[b % 2].astype(jnp.bfloat16)
            if b + 2 < N_DEV:
                xcopies[b + 2].start()
            for q in range(N_Q):
                qsl = pl.ds(q * m_q, m_q)
                rdma = pltpu.make_async_remote_copy(
                    src_ref=sendbuf.at[b, qsl, :],
                    dst_ref=comm_ref.at[my, qsl, :],
                    send_sem=send_sems.at[len(sends)],
                    recv_sem=recv_sems.at[my, q],
                    device_id=(tgt,),
                    device_id_type=pl.DeviceIdType.MESH,
                )
                rdma.start()
                sends.append(rdma)
        xcopies[3].wait()
        comm_ref[my] = xstage[1].astype(jnp.bfloat16)

        w_done = [False] * (2 * N_DEV)

        def ensure_w(block_i):
            for h in range(2):
                ci = 2 * block_i + h
                if not w_done[ci]:
                    wcopies[ci].wait()
                    wbf[ci] = wbuf[ci % 2].astype(jnp.bfloat16)
                    if ci + 2 < 2 * N_DEV:
                        wcopies[ci + 2].start()
                    w_done[ci] = True

        def do_quarter(block_i, src, q, is_first):
            qsl = pl.ds(q * m_q, m_q)
            for h in range(2):
                nsl = pl.ds(h * n_half, n_half)
                partial = jnp.dot(
                    comm_ref[src, qsl, :],
                    wbf[2 * block_i + h],
                    preferred_element_type=jnp.float32,
                )
                if is_first:
                    acc[qsl, nsl] = partial
                else:
                    acc[qsl, nsl] = acc[qsl, nsl] + partial

        ensure_w(0)
        for q in range(N_Q):
            do_quarter(0, my, q, True)

        outcopies = [
            pltpu.make_async_copy(
                acc.at[pl.ds(q * m_q, m_q), :],
                out_hbm.at[pl.ds(q * m_q, m_q), :],
                osem.at[q],
            )
            for q in range(N_Q)
        ]
        for q in range(N_Q):
            for block_i, src in ((1, right), (2, left), (3, diag)):
                ensure_w(block_i)
                qsl = pl.ds(q * m_q, m_q)
                recv = pltpu.make_async_remote_copy(
                    src_ref=comm_ref.at[src, qsl, :],
                    dst_ref=comm_ref.at[src, qsl, :],
                    send_sem=send_sems.at[0],
                    recv_sem=recv_sems.at[src, q],
                    device_id=(my,),
                    device_id_type=pl.DeviceIdType.MESH,
                )
                recv.wait_recv()
                do_quarter(block_i, src, q, False)
            outcopies[q].start()

        for oc in outcopies:
            oc.wait()
        for rdma in sends:
            rdma.wait_send()

    return pl.pallas_call(
        body,
        out_shape=jax.ShapeDtypeStruct((m_per, n), jnp.float32),
        in_specs=[
            pl.BlockSpec(memory_space=pl.ANY),
            pl.BlockSpec(memory_space=pl.ANY),
        ],
        out_specs=pl.BlockSpec(memory_space=pl.ANY),
        scratch_shapes=[
            pltpu.VMEM((N_DEV, m_per, k_per), jnp.bfloat16),
            pltpu.VMEM((N_DEV - 1, m_per, k_per), jnp.bfloat16),
            pltpu.VMEM((2, m_per, k_per), jnp.float32),
            pltpu.VMEM((2, m_per, n_half), jnp.float32),
            pltpu.VMEM((2 * N_DEV, m_per, n_half), jnp.bfloat16),
            pltpu.VMEM((m_per, n), jnp.float32),
            pltpu.SemaphoreType.DMA(((N_DEV - 1) * N_Q,)),
            pltpu.SemaphoreType.DMA((N_DEV, N_Q)),
            pltpu.SemaphoreType.DMA((2,)),
            pltpu.SemaphoreType.DMA((2,)),
            pltpu.SemaphoreType.DMA((N_Q,)),
        ],
        compiler_params=pltpu.CompilerParams(
            collective_id=0,
            vmem_limit_bytes=100 * 1024 * 1024,
        ),
    )(x, w_mat)


# device time: 62765 ns/iter; 1.4631x vs baseline; 1.0164x over previous
import jax
import jax.numpy as jnp
from jax import lax
from jax.experimental import pallas as pl
from jax.experimental.pallas import tpu as pltpu

N_DEV = 4
N_Q = 4


def kernel(x, w_mat):
    k_full, k_per = x.shape
    _, n = w_mat.shape
    m_per = k_full // N_DEV
    m_q = m_per // N_Q
    n_half = n // 2
    assert m_per == k_per, (x.shape, w_mat.shape)

    def body(x_hbm, w_hbm, out_hbm, comm_ref, sendbuf, xstage, xlocal,
             wbuf, wbf, acc, send_sems, recv_sems, wsem, xsem, xlsem,
             osem):
        my = lax.axis_index("i")
        right = (my + 1) % N_DEV
        left = (my + 3) % N_DEV
        diag = (my + 2) % N_DEV

        barrier = pltpu.get_barrier_semaphore()
        for d in range(1, N_DEV):
            pl.semaphore_signal(
                barrier,
                inc=1,
                device_id=((my + d) % N_DEV,),
                device_id_type=pl.DeviceIdType.MESH,
            )

        conv_order = [diag, right, left]
        xcopies = [
            pltpu.make_async_copy(
                x_hbm.at[pl.ds(blk * m_per + q * m_q, m_q), :],
                xstage.at[(4 * b + q) % 2],
                xsem.at[(4 * b + q) % 2],
            )
            for b, blk in enumerate(conv_order)
            for q in range(N_Q)
        ]
        xcopies[0].start()
        xcopies[1].start()
        xlocal_cpy = pltpu.make_async_copy(
            x_hbm.at[pl.ds(my * m_per, m_per), :], xlocal, xlsem
        )
        xlocal_cpy.start()

        proc_order = [my, right, left, diag]
        wcopies = [
            pltpu.make_async_copy(
                w_hbm.at[pl.ds(kblk * m_per, m_per),
                         pl.ds(h * n_half, n_half)],
                wbuf.at[ci % 2],
                wsem.at[ci % 2],
            )
            for ci, (kblk, h) in enumerate(
                (kblk, h) for kblk in proc_order for h in range(2)
            )
        ]
        wcopies[0].start()
        wcopies[1].start()

        pl.semaphore_wait(barrier, N_DEV - 1)

        sends = []
        for b in range(N_DEV - 1):
            tgt = conv_order[b]
            for q in range(N_Q):
                qi = 4 * b + q
                qsl = pl.ds(q * m_q, m_q)
                xcopies[qi].wait()
                sendbuf[b, qsl, :] = xstage[qi % 2].astype(jnp.bfloat16)
                if qi + 2 < len(xcopies):
                    xcopies[qi + 2].start()
                rdma = pltpu.make_async_remote_copy(
                    src_ref=sendbuf.at[b, qsl, :],
                    dst_ref=comm_ref.at[my, qsl, :],
                    send_sem=send_sems.at[len(sends)],
                    recv_sem=recv_sems.at[my, q],
                    device_id=(tgt,),
                    device_id_type=pl.DeviceIdType.MESH,
                )
                rdma.start()
                sends.append(rdma)
        xlocal_cpy.wait()
        comm_ref[my] = xlocal[...].astype(jnp.bfloat16)

        w_done = [False] * (2 * N_DEV)

        def ensure_w(block_i):
            for h in range(2):
                ci = 2 * block_i + h
                if not w_done[ci]:
                    wcopies[ci].wait()
                    wbf[ci] = wbuf[ci % 2].astype(jnp.bfloat16)
                    if ci + 2 < 2 * N_DEV:
                        wcopies[ci + 2].start()
                    w_done[ci] = True

        def do_quarter(block_i, src, q, is_first):
            qsl = pl.ds(q * m_q, m_q)
            for h in range(2):
                nsl = pl.ds(h * n_half, n_half)
                partial = jnp.dot(
                    comm_ref[src, qsl, :],
                    wbf[2 * block_i + h],
                    preferred_element_type=jnp.float32,
                )
                if is_first:
                    acc[qsl, nsl] = partial
                else:
                    acc[qsl, nsl] = acc[qsl, nsl] + partial

        ensure_w(0)
        for q in range(N_Q):
            do_quarter(0, my, q, True)

        outcopies = [
            pltpu.make_async_copy(
                acc.at[pl.ds(q * m_q, m_q), :],
                out_hbm.at[pl.ds(q * m_q, m_q), :],
                osem.at[q],
            )
            for q in range(N_Q)
        ]
        for q in range(N_Q):
            for block_i, src in ((1, right), (2, left), (3, diag)):
                ensure_w(block_i)
                qsl = pl.ds(q * m_q, m_q)
                recv = pltpu.make_async_remote_copy(
                    src_ref=comm_ref.at[src, qsl, :],
                    dst_ref=comm_ref.at[src, qsl, :],
                    send_sem=send_sems.at[0],
                    recv_sem=recv_sems.at[src, q],
                    device_id=(my,),
                    device_id_type=pl.DeviceIdType.MESH,
                )
                recv.wait_recv()
                do_quarter(block_i, src, q, False)
            outcopies[q].start()

        for oc in outcopies:
            oc.wait()
        for rdma in sends:
            rdma.wait_send()

    return pl.pallas_call(
        body,
        out_shape=jax.ShapeDtypeStruct((m_per, n), jnp.float32),
        in_specs=[
            pl.BlockSpec(memory_space=pl.ANY),
            pl.BlockSpec(memory_space=pl.ANY),
        ],
        out_specs=pl.BlockSpec(memory_space=pl.ANY),
        scratch_shapes=[
            pltpu.VMEM((N_DEV, m_per, k_per), jnp.bfloat16),
            pltpu.VMEM((N_DEV - 1, m_per, k_per), jnp.bfloat16),
            pltpu.VMEM((2, m_q, k_per), jnp.float32),
            pltpu.VMEM((m_per, k_per), jnp.float32),
            pltpu.VMEM((2, m_per, n_half), jnp.float32),
            pltpu.VMEM((2 * N_DEV, m_per, n_half), jnp.bfloat16),
            pltpu.VMEM((m_per, n), jnp.float32),
            pltpu.SemaphoreType.DMA(((N_DEV - 1) * N_Q,)),
            pltpu.SemaphoreType.DMA((N_DEV, N_Q)),
            pltpu.SemaphoreType.DMA((2,)),
            pltpu.SemaphoreType.DMA((2,)),
            pltpu.SemaphoreType.DMA,
            pltpu.SemaphoreType.DMA((N_Q,)),
        ],
        compiler_params=pltpu.CompilerParams(
            collective_id=0,
            vmem_limit_bytes=100 * 1024 * 1024,
        ),
    )(x, w_mat)


# device time: 60807 ns/iter; 1.5102x vs baseline; 1.0322x over previous
import jax
import jax.numpy as jnp
from jax import lax
from jax.experimental import pallas as pl
from jax.experimental.pallas import tpu as pltpu

N_DEV = 4
N_Q = 8


def kernel(x, w_mat):
    k_full, k_per = x.shape
    _, n = w_mat.shape
    m_per = k_full // N_DEV
    m_q = m_per // N_Q
    n_half = n // 2
    assert m_per == k_per, (x.shape, w_mat.shape)

    def body(x_hbm, w_hbm, out_hbm, comm_ref, sendbuf, xstage, xlocal,
             wbuf, wbf, acc, send_sems, recv_sems, wsem, xsem, xlsem,
             osem):
        my = lax.axis_index("i")
        right = (my + 1) % N_DEV
        left = (my + 3) % N_DEV
        diag = (my + 2) % N_DEV

        barrier = pltpu.get_barrier_semaphore()
        for d in range(1, N_DEV):
            pl.semaphore_signal(
                barrier,
                inc=1,
                device_id=((my + d) % N_DEV,),
                device_id_type=pl.DeviceIdType.MESH,
            )

        conv_order = [diag, right, left]
        xcopies = [
            pltpu.make_async_copy(
                x_hbm.at[pl.ds(blk * m_per + q * m_q, m_q), :],
                xstage.at[(N_Q * b + q) % 2],
                xsem.at[(N_Q * b + q) % 2],
            )
            for b, blk in enumerate(conv_order)
            for q in range(N_Q)
        ]
        xcopies[0].start()
        xcopies[1].start()
        xlocal_cpy = pltpu.make_async_copy(
            x_hbm.at[pl.ds(my * m_per, m_per), :], xlocal, xlsem
        )
        xlocal_cpy.start()

        proc_order = [my, right, left, diag]
        wcopies = [
            pltpu.make_async_copy(
                w_hbm.at[pl.ds(kblk * m_per, m_per),
                         pl.ds(h * n_half, n_half)],
                wbuf.at[ci % 2],
                wsem.at[ci % 2],
            )
            for ci, (kblk, h) in enumerate(
                (kblk, h) for kblk in proc_order for h in range(2)
            )
        ]
        wcopies[0].start()
        wcopies[1].start()

        pl.semaphore_wait(barrier, N_DEV - 1)

        sends = []
        for b in range(N_DEV - 1):
            tgt = conv_order[b]
            for q in range(N_Q):
                qi = N_Q * b + q
                qsl = pl.ds(q * m_q, m_q)
                xcopies[qi].wait()
                sendbuf[b, qsl, :] = xstage[qi % 2].astype(jnp.bfloat16)
                if qi + 2 < len(xcopies):
                    xcopies[qi + 2].start()
                rdma = pltpu.make_async_remote_copy(
                    src_ref=sendbuf.at[b, qsl, :],
                    dst_ref=comm_ref.at[my, qsl, :],
                    send_sem=send_sems.at[len(sends)],
                    recv_sem=recv_sems.at[my, q],
                    device_id=(tgt,),
                    device_id_type=pl.DeviceIdType.MESH,
                )
                rdma.start()
                sends.append(rdma)
        xlocal_cpy.wait()
        comm_ref[my] = xlocal[...].astype(jnp.bfloat16)

        w_done = [False] * (2 * N_DEV)

        def ensure_w(block_i):
            for h in range(2):
                ci = 2 * block_i + h
                if not w_done[ci]:
                    wcopies[ci].wait()
                    wbf[ci] = wbuf[ci % 2].astype(jnp.bfloat16)
                    if ci + 2 < 2 * N_DEV:
                        wcopies[ci + 2].start()
                    w_done[ci] = True

        def do_quarter(block_i, src, q, is_first):
            qsl = pl.ds(q * m_q, m_q)
            for h in range(2):
                nsl = pl.ds(h * n_half, n_half)
                partial = jnp.dot(
                    comm_ref[src, qsl, :],
                    wbf[2 * block_i + h],
                    preferred_element_type=jnp.float32,
                )
                if is_first:
                    acc[qsl, nsl] = partial
                else:
                    acc[qsl, nsl] = acc[qsl, nsl] + partial

        ensure_w(0)
        for q in range(N_Q):
            do_quarter(0, my, q, True)

        outcopies = [
            pltpu.make_async_copy(
                acc.at[pl.ds(q * m_q, m_q), :],
                out_hbm.at[pl.ds(q * m_q, m_q), :],
                osem.at[q],
            )
            for q in range(N_Q)
        ]
        for q in range(N_Q):
            for block_i, src in ((1, right), (2, left), (3, diag)):
                ensure_w(block_i)
                qsl = pl.ds(q * m_q, m_q)
                recv = pltpu.make_async_remote_copy(
                    src_ref=comm_ref.at[src, qsl, :],
                    dst_ref=comm_ref.at[src, qsl, :],
                    send_sem=send_sems.at[0],
                    recv_sem=recv_sems.at[src, q],
                    device_id=(my,),
                    device_id_type=pl.DeviceIdType.MESH,
                )
                recv.wait_recv()
                do_quarter(block_i, src, q, False)
            outcopies[q].start()

        for oc in outcopies:
            oc.wait()
        for rdma in sends:
            rdma.wait_send()

    return pl.pallas_call(
        body,
        out_shape=jax.ShapeDtypeStruct((m_per, n), jnp.float32),
        in_specs=[
            pl.BlockSpec(memory_space=pl.ANY),
            pl.BlockSpec(memory_space=pl.ANY),
        ],
        out_specs=pl.BlockSpec(memory_space=pl.ANY),
        scratch_shapes=[
            pltpu.VMEM((N_DEV, m_per, k_per), jnp.bfloat16),
            pltpu.VMEM((N_DEV - 1, m_per, k_per), jnp.bfloat16),
            pltpu.VMEM((2, m_q, k_per), jnp.float32),
            pltpu.VMEM((m_per, k_per), jnp.float32),
            pltpu.VMEM((2, m_per, n_half), jnp.float32),
            pltpu.VMEM((2 * N_DEV, m_per, n_half), jnp.bfloat16),
            pltpu.VMEM((m_per, n), jnp.float32),
            pltpu.SemaphoreType.DMA(((N_DEV - 1) * N_Q,)),
            pltpu.SemaphoreType.DMA((N_DEV, N_Q)),
            pltpu.SemaphoreType.DMA((2,)),
            pltpu.SemaphoreType.DMA((2,)),
            pltpu.SemaphoreType.DMA,
            pltpu.SemaphoreType.DMA((N_Q,)),
        ],
        compiler_params=pltpu.CompilerParams(
            collective_id=0,
            vmem_limit_bytes=100 * 1024 * 1024,
        ),
    )(x, w_mat)
